# baseline (speedup 1.0000x reference)
import functools

import jax
import jax.numpy as jnp
from jax import lax
from jax.experimental import pallas as pl
from jax.experimental.pallas import tpu as pltpu
from jax.experimental.pallas import tpu_sc as plsc

D_MODEL = 768
LN_EPS = 1e-12


def _word_gather_sc(ids, word_emb, *, n_tokens):
    info = plsc.get_sparse_core_info()
    n_workers = info.num_cores * info.num_subcores
    tpw = n_tokens // n_workers
    chunk = 64
    n_chunks = tpw // chunk
    mesh = plsc.VectorSubcoreMesh(core_axis_name="c", subcore_axis_name="s")

    @functools.partial(
        pl.kernel,
        out_type=jax.ShapeDtypeStruct((n_tokens, D_MODEL), jnp.float32),
        mesh=mesh,
        scratch_types=[
            pltpu.VMEM((tpw,), jnp.int32),
            pltpu.VMEM((2, chunk, D_MODEL), jnp.float32),
            pltpu.SemaphoreType.DMA,
            pltpu.SemaphoreType.DMA,
            pltpu.SemaphoreType.DMA,
            pltpu.SemaphoreType.DMA,
        ],
    )
    def gather_kernel(ids_hbm, word_hbm, out_hbm, idx_v, rows_v,
                      sem_g0, sem_g1, sem_w0, sem_w1):
        wid = lax.axis_index("s") * info.num_cores + lax.axis_index("c")
        base = wid * tpw
        sems_g = (sem_g0, sem_g1)
        sems_w = (sem_w0, sem_w1)
        pltpu.sync_copy(ids_hbm.at[pl.ds(base, tpw)], idx_v)

        def start_gather(c, buf):
            return pltpu.async_copy(
                word_hbm.at[idx_v.at[pl.ds(c * chunk, chunk)]],
                rows_v.at[buf], sems_g[buf])

        def start_write(c, buf):
            return pltpu.async_copy(
                rows_v.at[buf], out_hbm.at[pl.ds(base + c * chunk, chunk)],
                sems_w[buf])

        gathers = [start_gather(0, 0), None]
        writes = [None, None]
        for c in range(n_chunks):
            buf = c % 2
            gathers[buf].wait()
            writes[buf] = start_write(c, buf)
            nc = c + 1
            if nc < n_chunks:
                nbuf = nc % 2
                if writes[nbuf] is not None:
                    writes[nbuf].wait()
                gathers[nbuf] = start_gather(nc, nbuf)
        for buf in (0, 1):
            if writes[buf] is not None:
                writes[buf].wait()

    return gather_kernel(ids, word_emb)


def _add_ln_tc(x, tts3, pos_emb, type_emb, gamma, beta, out_buf, *,
               seq_len, blk, row_off):
    n = x.shape[0]
    pos_blocks = seq_len // blk
    batch = n // seq_len
    off_blocks = row_off // blk

    def body(x_ref, tt_ref, pos_ref, typ_ref, g_ref, b_ref, buf_ref, o_ref):
        del buf_ref
        ttf = tt_ref[0].astype(jnp.float32).reshape(blk, 1)
        t0 = typ_ref[0:1, :]
        t1 = typ_ref[1:2, :]
        xb = x_ref[...] + pos_ref[...] + t0 + ttf * (t1 - t0)
        mean = jnp.mean(xb, axis=-1, keepdims=True)
        xc = xb - mean
        var = jnp.mean(xc * xc, axis=-1, keepdims=True)
        o_ref[...] = xc * lax.rsqrt(var + LN_EPS) * g_ref[...] + b_ref[...]

    return pl.pallas_call(
        body,
        grid=(pos_blocks, batch),
        in_specs=[
            pl.BlockSpec((blk, D_MODEL), lambda j, i: (i * pos_blocks + j, 0)),
            pl.BlockSpec((1, 1, blk), lambda j, i: (i * pos_blocks + j, 0, 0)),
            pl.BlockSpec((blk, D_MODEL), lambda j, i: (j, 0)),
            pl.BlockSpec((2, D_MODEL), lambda j, i: (0, 0)),
            pl.BlockSpec((1, D_MODEL), lambda j, i: (0, 0)),
            pl.BlockSpec((1, D_MODEL), lambda j, i: (0, 0)),
            pl.BlockSpec(memory_space=pl.ANY),
        ],
        out_specs=pl.BlockSpec(
            (blk, D_MODEL),
            lambda j, i: (off_blocks + i * pos_blocks + j, 0)),
        out_shape=jax.ShapeDtypeStruct(out_buf.shape, jnp.float32),
        input_output_aliases={6: 0},
    )(x, tts3, pos_emb, type_emb, gamma.reshape(1, D_MODEL),
      beta.reshape(1, D_MODEL), out_buf)


def kernel(input_ids, token_type_ids, word_emb, pos_emb, type_emb, ln_gamma, ln_beta):
    b, s = input_ids.shape
    n = b * s
    blk = 512
    half = n // 2
    ids = input_ids.reshape(n).astype(jnp.int32)
    tts3 = token_type_ids.reshape(n // blk, 1, blk).astype(jnp.int32)
    n_tt = tts3.shape[0]

    g0 = _word_gather_sc(ids[:half], word_emb, n_tokens=half)
    g1 = _word_gather_sc(ids[half:], word_emb, n_tokens=half)

    buf = jnp.empty((n, D_MODEL), dtype=jnp.float32)
    buf = _add_ln_tc(g0, tts3[: n_tt // 2], pos_emb, type_emb,
                     ln_gamma, ln_beta, buf, seq_len=s, blk=blk, row_off=0)
    buf = _add_ln_tc(g1, tts3[n_tt // 2:], pos_emb, type_emb,
                     ln_gamma, ln_beta, buf, seq_len=s, blk=blk, row_off=half)
    return buf.reshape(b, s, D_MODEL)


# speedup vs baseline: 1.1792x; 1.1792x over previous
import functools

import jax
import jax.numpy as jnp
from jax import lax
from jax.experimental import pallas as pl
from jax.experimental.pallas import tpu as pltpu
from jax.experimental.pallas import tpu_sc as plsc

D_MODEL = 768
LN_EPS = 1e-12


def _word_gather_sc(ids, word_emb, *, n_tokens):
    info = plsc.get_sparse_core_info()
    n_workers = info.num_cores * info.num_subcores
    tpw = n_tokens // n_workers
    chunk = 64
    n_chunks = tpw // chunk
    mesh = plsc.VectorSubcoreMesh(core_axis_name="c", subcore_axis_name="s")

    @functools.partial(
        pl.kernel,
        out_type=jax.ShapeDtypeStruct((n_tokens, D_MODEL), jnp.float32),
        mesh=mesh,
        scratch_types=[
            pltpu.VMEM((tpw,), jnp.int32),
            pltpu.VMEM((2, chunk, D_MODEL), jnp.float32),
            pltpu.SemaphoreType.DMA,
            pltpu.SemaphoreType.DMA,
            pltpu.SemaphoreType.DMA,
            pltpu.SemaphoreType.DMA,
        ],
    )
    def gather_kernel(ids_hbm, word_hbm, out_hbm, idx_v, rows_v,
                      sem_g0, sem_g1, sem_w0, sem_w1):
        wid = lax.axis_index("s") * info.num_cores + lax.axis_index("c")
        base = wid * tpw
        sems_g = (sem_g0, sem_g1)
        sems_w = (sem_w0, sem_w1)
        pltpu.sync_copy(ids_hbm.at[pl.ds(base, tpw)], idx_v)

        def start_gather(c, buf):
            return pltpu.async_copy(
                word_hbm.at[idx_v.at[pl.ds(c * chunk, chunk)]],
                rows_v.at[buf], sems_g[buf])

        def start_write(c, buf):
            return pltpu.async_copy(
                rows_v.at[buf], out_hbm.at[pl.ds(base + c * chunk, chunk)],
                sems_w[buf])

        gathers = [start_gather(0, 0), None]
        writes = [None, None]
        for c in range(n_chunks):
            buf = c % 2
            gathers[buf].wait()
            writes[buf] = start_write(c, buf)
            nc = c + 1
            if nc < n_chunks:
                nbuf = nc % 2
                if writes[nbuf] is not None:
                    writes[nbuf].wait()
                gathers[nbuf] = start_gather(nc, nbuf)
        for buf in (0, 1):
            if writes[buf] is not None:
                writes[buf].wait()

    return gather_kernel(ids, word_emb)


def _add_ln_tc(x, tts3, pos_emb, type_emb, gamma, beta, out_buf, *,
               seq_len, blk, row_off, n_total):
    n = x.shape[0]
    pos_blocks = seq_len // blk
    batch = n // seq_len
    off_blocks = row_off // blk

    def body(x_ref, tt_ref, pos_ref, typ_ref, g_ref, b_ref, *rest):
        o_ref = rest[-1]
        ttf = tt_ref[0].astype(jnp.float32).reshape(blk, 1)
        t0 = typ_ref[0:1, :]
        t1 = typ_ref[1:2, :]
        xb = x_ref[...] + pos_ref[...] + t0 + ttf * (t1 - t0)
        mean = jnp.mean(xb, axis=-1, keepdims=True)
        xc = xb - mean
        var = jnp.mean(xc * xc, axis=-1, keepdims=True)
        o_ref[...] = xc * lax.rsqrt(var + LN_EPS) * g_ref[...] + b_ref[...]

    in_specs = [
        pl.BlockSpec((blk, D_MODEL), lambda j, i: (i * pos_blocks + j, 0)),
        pl.BlockSpec((1, 1, blk), lambda j, i: (i * pos_blocks + j, 0, 0)),
        pl.BlockSpec((blk, D_MODEL), lambda j, i: (j, 0)),
        pl.BlockSpec((2, D_MODEL), lambda j, i: (0, 0)),
        pl.BlockSpec((1, D_MODEL), lambda j, i: (0, 0)),
        pl.BlockSpec((1, D_MODEL), lambda j, i: (0, 0)),
    ]
    args = [x, tts3, pos_emb, type_emb, gamma.reshape(1, D_MODEL),
            beta.reshape(1, D_MODEL)]
    aliases = {}
    if out_buf is not None:
        in_specs.append(pl.BlockSpec(memory_space=pl.ANY))
        args.append(out_buf)
        aliases = {6: 0}

    return pl.pallas_call(
        body,
        grid=(pos_blocks, batch),
        in_specs=in_specs,
        out_specs=pl.BlockSpec(
            (blk, D_MODEL),
            lambda j, i: (off_blocks + i * pos_blocks + j, 0)),
        out_shape=jax.ShapeDtypeStruct((n_total, D_MODEL), jnp.float32),
        input_output_aliases=aliases,
    )(*args)


def kernel(input_ids, token_type_ids, word_emb, pos_emb, type_emb, ln_gamma, ln_beta):
    b, s = input_ids.shape
    n = b * s
    blk = 512
    half = n // 2
    ids = input_ids.reshape(n).astype(jnp.int32)
    tts3 = token_type_ids.reshape(n // blk, 1, blk).astype(jnp.int32)
    n_tt = tts3.shape[0]

    g0 = _word_gather_sc(ids[:half], word_emb, n_tokens=half)
    g1 = _word_gather_sc(ids[half:], word_emb, n_tokens=half)

    buf = _add_ln_tc(g0, tts3[: n_tt // 2], pos_emb, type_emb,
                     ln_gamma, ln_beta, None, seq_len=s, blk=blk,
                     row_off=0, n_total=n)
    buf = _add_ln_tc(g1, tts3[n_tt // 2:], pos_emb, type_emb,
                     ln_gamma, ln_beta, buf, seq_len=s, blk=blk,
                     row_off=half, n_total=n)
    return buf.reshape(b, s, D_MODEL)


# speedup vs baseline: 1.2155x; 1.0308x over previous
import functools

import jax
import jax.numpy as jnp
from jax import lax
from jax.experimental import pallas as pl
from jax.experimental.pallas import tpu as pltpu
from jax.experimental.pallas import tpu_sc as plsc

D_MODEL = 768
LN_EPS = 1e-12


def _word_gather_sc(ids, word_emb, *, n_tokens):
    info = plsc.get_sparse_core_info()
    n_workers = info.num_cores * info.num_subcores
    tpw = n_tokens // n_workers
    chunk = 64
    n_chunks = tpw // chunk
    mesh = plsc.VectorSubcoreMesh(core_axis_name="c", subcore_axis_name="s")

    @functools.partial(
        pl.kernel,
        out_type=jax.ShapeDtypeStruct((n_tokens, D_MODEL), jnp.float32),
        mesh=mesh,
        scratch_types=[
            pltpu.VMEM((tpw,), jnp.int32),
            pltpu.VMEM((2, chunk, D_MODEL), jnp.float32),
            pltpu.SemaphoreType.DMA,
            pltpu.SemaphoreType.DMA,
            pltpu.SemaphoreType.DMA,
            pltpu.SemaphoreType.DMA,
        ],
    )
    def gather_kernel(ids_hbm, word_hbm, out_hbm, idx_v, rows_v,
                      sem_g0, sem_g1, sem_w0, sem_w1):
        wid = lax.axis_index("s") * info.num_cores + lax.axis_index("c")
        base = wid * tpw
        sems_g = (sem_g0, sem_g1)
        sems_w = (sem_w0, sem_w1)
        pltpu.sync_copy(ids_hbm.at[pl.ds(base, tpw)], idx_v)

        def start_gather(c, buf):
            return pltpu.async_copy(
                word_hbm.at[idx_v.at[pl.ds(c * chunk, chunk)]],
                rows_v.at[buf], sems_g[buf])

        def start_write(c, buf):
            return pltpu.async_copy(
                rows_v.at[buf], out_hbm.at[pl.ds(base + c * chunk, chunk)],
                sems_w[buf])

        gathers = [start_gather(0, 0), None]
        writes = [None, None]
        for c in range(n_chunks):
            buf = c % 2
            gathers[buf].wait()
            writes[buf] = start_write(c, buf)
            nc = c + 1
            if nc < n_chunks:
                nbuf = nc % 2
                if writes[nbuf] is not None:
                    writes[nbuf].wait()
                gathers[nbuf] = start_gather(nc, nbuf)
        for buf in (0, 1):
            if writes[buf] is not None:
                writes[buf].wait()

    return gather_kernel(ids, word_emb)


def _add_ln_tc(x, tts3, pos_emb, type_emb, gamma, beta, out_buf, *,
               seq_len, blk, row_off, n_total):
    n = x.shape[0]
    pos_blocks = seq_len // blk
    batch = n // seq_len
    off_blocks = row_off // blk

    def body(x_ref, tt_ref, pos_ref, typ_ref, g_ref, b_ref, *rest):
        o_ref = rest[-1]
        ttf = tt_ref[0].astype(jnp.float32).reshape(blk, 1)
        t0 = typ_ref[0:1, :]
        t1 = typ_ref[1:2, :]
        xb = x_ref[...] + pos_ref[...] + t0 + ttf * (t1 - t0)
        mean = jnp.mean(xb, axis=-1, keepdims=True)
        xc = xb - mean
        var = jnp.mean(xc * xc, axis=-1, keepdims=True)
        o_ref[...] = xc * lax.rsqrt(var + LN_EPS) * g_ref[...] + b_ref[...]

    in_specs = [
        pl.BlockSpec((blk, D_MODEL), lambda j, i: (i * pos_blocks + j, 0)),
        pl.BlockSpec((1, 1, blk), lambda j, i: (i * pos_blocks + j, 0, 0)),
        pl.BlockSpec((blk, D_MODEL), lambda j, i: (j, 0)),
        pl.BlockSpec((2, D_MODEL), lambda j, i: (0, 0)),
        pl.BlockSpec((1, D_MODEL), lambda j, i: (0, 0)),
        pl.BlockSpec((1, D_MODEL), lambda j, i: (0, 0)),
    ]
    args = [x, tts3, pos_emb, type_emb, gamma.reshape(1, D_MODEL),
            beta.reshape(1, D_MODEL)]
    aliases = {}
    if out_buf is not None:
        in_specs.append(pl.BlockSpec(memory_space=pl.ANY))
        args.append(out_buf)
        aliases = {6: 0}

    return pl.pallas_call(
        body,
        grid=(pos_blocks, batch),
        in_specs=in_specs,
        out_specs=pl.BlockSpec(
            (blk, D_MODEL),
            lambda j, i: (off_blocks + i * pos_blocks + j, 0)),
        out_shape=jax.ShapeDtypeStruct((n_total, D_MODEL), jnp.float32),
        input_output_aliases=aliases,
    )(*args)


def kernel(input_ids, token_type_ids, word_emb, pos_emb, type_emb, ln_gamma, ln_beta):
    b, s = input_ids.shape
    n = b * s
    blk = 1024
    half = n // 2
    ids = input_ids.reshape(n).astype(jnp.int32)
    tts3 = token_type_ids.reshape(n // blk, 1, blk).astype(jnp.int32)
    n_tt = tts3.shape[0]

    g0 = _word_gather_sc(ids[:half], word_emb, n_tokens=half)
    g1 = _word_gather_sc(ids[half:], word_emb, n_tokens=half)

    buf = _add_ln_tc(g0, tts3[: n_tt // 2], pos_emb, type_emb,
                     ln_gamma, ln_beta, None, seq_len=s, blk=blk,
                     row_off=0, n_total=n)
    buf = _add_ln_tc(g1, tts3[n_tt // 2:], pos_emb, type_emb,
                     ln_gamma, ln_beta, buf, seq_len=s, blk=blk,
                     row_off=half, n_total=n)
    return buf.reshape(b, s, D_MODEL)
